# baseline (speedup 1.0000x reference)
import functools

import jax
import jax.numpy as jnp
from jax import lax
from jax.experimental import pallas as pl
from jax.experimental.pallas import tpu as pltpu
from jax.experimental.pallas import tpu_sc as plsc

DIM = 32
NI = 16384
NJ = 50
NJP = 56
NBLK = 250000
NC, NS = 2, 16
NW = NC * NS
IW = NI // NW
HALF = 256


@functools.partial(
    pl.kernel,
    out_type=jax.ShapeDtypeStruct((NJ, DIM, NI), jnp.float32),
    mesh=plsc.VectorSubcoreMesh(core_axis_name="c", subcore_axis_name="s"),
    scratch_types=[
        pltpu.VMEM((16, IW), jnp.int32),
        pltpu.VMEM((HALF,), jnp.int32),
        pltpu.VMEM((HALF,), jnp.int32),
        pltpu.VMEM((HALF,), jnp.int32),
        pltpu.VMEM((HALF,), jnp.int32),
        pltpu.VMEM((HALF, 128), jnp.float32),
        pltpu.VMEM((HALF, 128), jnp.float32),
        pltpu.VMEM((DIM, HALF), jnp.float32),
        pltpu.VMEM((DIM, HALF), jnp.float32),
        pltpu.SemaphoreType.DMA,
        pltpu.SemaphoreType.DMA,
        pltpu.SemaphoreType.DMA,
        pltpu.SemaphoreType.DMA,
    ],
    compiler_params=pltpu.CompilerParams(
        use_tc_tiling_on_sc=True, needs_layout_passes=False
    ),
)
def _emb_kernel(idx_hbm, table_hbm, out_hbm, idx_blk, bidxA, bidxB,
                qoffA, qoffB, bigA, bigB, outA, outB,
                gsemA, gsemB, wsemA, wsemB):
    wid = lax.axis_index("s") * NC + lax.axis_index("c")
    i0 = pl.multiple_of(wid * IW, 128)

    def load_idx_rows(j2):
        half = (j2 >> 3) & 1
        pltpu.sync_copy(
            idx_hbm.at[pl.ds(pl.multiple_of(j2, 8), 8), pl.ds(i0, IW)],
            idx_blk.at[pl.ds(pl.multiple_of(half * 8, 8), 8), :],
        )

    def prep(j, h, bidx_v, qoff_v):
        row = j & 15

        def body(k, carry):
            v = idx_blk[row, pl.ds(h * HALF + k * 16, 16)]
            bidx_v[pl.ds(k * 16, 16)] = lax.shift_right_logical(v, 2)
            qoff_v[pl.ds(k * 16, 16)] = lax.shift_left(v & 3, 5)
            return carry

        lax.fori_loop(0, HALF // 16, body, 0)

    def extract(big_v, qoff_v, out_v):
        def body(k, carry):
            row_v = lax.iota(jnp.int32, 16) + k * 16
            colb = qoff_v[pl.ds(k * 16, 16)]
            for c in range(DIM):
                out_v[c, pl.ds(k * 16, 16)] = plsc.load_gather(
                    big_v, [row_v, colb + c]
                )
            return carry

        lax.fori_loop(0, HALF // 16, body, 0)

    def fire(bidx_v, big_v, gsem):
        pltpu.async_copy(table_hbm.at[bidx_v], big_v, gsem)

    def wait_g(bidx_v, big_v, gsem):
        pltpu.make_async_copy(table_hbm.at[bidx_v], big_v, gsem).wait()

    def write(j, h, out_v, wsem):
        pltpu.async_copy(
            out_v, out_hbm.at[j, :, pl.ds(i0 + h * HALF, HALF)], wsem
        )

    def wait_w(out_v, wsem):
        pltpu.make_async_copy(
            out_v, out_hbm.at[0, :, pl.ds(i0, HALF)], wsem
        ).wait()

    load_idx_rows(0)
    prep(0, 0, bidxA, qoffA)
    fire(bidxA, bigA, gsemA)
    prep(0, 1, bidxB, qoffB)
    fire(bidxB, bigB, gsemB)
    pltpu.async_copy(outA, out_hbm.at[0, :, pl.ds(i0, HALF)], wsemA)
    pltpu.async_copy(outB, out_hbm.at[0, :, pl.ds(i0 + HALF, HALF)], wsemB)

    def iter_t(t, carry):
        @pl.when(((t + 1) & 7) == 0)
        def _():
            load_idx_rows(t + 1)

        wait_w(outA, wsemA)
        wait_g(bidxA, bigA, gsemA)
        extract(bigA, qoffA, outA)
        prep(t + 1, 0, bidxA, qoffA)
        fire(bidxA, bigA, gsemA)
        write(t, 0, outA, wsemA)
        wait_w(outB, wsemB)
        wait_g(bidxB, bigB, gsemB)
        extract(bigB, qoffB, outB)
        prep(t + 1, 1, bidxB, qoffB)
        fire(bidxB, bigB, gsemB)
        write(t, 1, outB, wsemB)
        return carry

    lax.fori_loop(0, NJ, iter_t, 0)

    wait_g(bidxA, bigA, gsemA)
    wait_g(bidxB, bigB, gsemB)
    wait_w(outA, wsemA)
    wait_w(outB, wsemB)


def kernel(states, table):
    statesT = jnp.swapaxes(states, 0, 1)
    statesT_p = jnp.pad(statesT, ((0, NJP - NJ), (0, 0)))
    table128 = table.reshape(NBLK, 128)
    outT = _emb_kernel(statesT_p, table128)
    return jnp.transpose(outT, (2, 0, 1))


# speedup vs baseline: 1.0482x; 1.0482x over previous
import functools

import jax
import jax.numpy as jnp
from jax import lax
from jax.experimental import pallas as pl
from jax.experimental.pallas import tpu as pltpu
from jax.experimental.pallas import tpu_sc as plsc

DIM = 32
NI = 16384
NJ = 50
NJP = 56
NBLK = 250000
NC, NS = 2, 16
NW = NC * NS
IW = NI // NW
HALF = 256


NFULL = (1000000 // 128) * 128
NBF = NFULL // 128
NTAIL = (1000000 - NFULL) * DIM // 128


@functools.partial(
    pl.kernel,
    out_type=jax.ShapeDtypeStruct((NBLK, 128), jnp.float32),
    mesh=plsc.VectorSubcoreMesh(core_axis_name="c", subcore_axis_name="s"),
    scratch_types=[
        pltpu.VMEM((DIM, 128), jnp.float32),
        pltpu.VMEM((DIM, 128), jnp.float32),
        pltpu.VMEM((32, 128), jnp.float32),
        pltpu.VMEM((32, 128), jnp.float32),
        pltpu.SemaphoreType.DMA,
        pltpu.SemaphoreType.DMA,
        pltpu.SemaphoreType.DMA,
        pltpu.SemaphoreType.DMA,
    ],
    compiler_params=pltpu.CompilerParams(
        use_tc_tiling_on_sc=True, needs_layout_passes=False
    ),
)
def _relayout_kernel(tableT_hbm, tail_hbm, out_hbm, blk0, blk1, rows0, rows1,
                     rsem0, rsem1, wsem0, wsem1):
    wid = lax.axis_index("s") * NC + lax.axis_index("c")
    blk = (blk0, blk1)
    rows = (rows0, rows1)
    rsem = (rsem0, rsem1)
    wsem = (wsem0, wsem1)

    @pl.when(wid == 0)
    def _():
        pltpu.sync_copy(tail_hbm, out_hbm.at[pl.ds(NBLK - NTAIL, NTAIL), :])

    def transpose(par):
        def body(rr, carry):
            for m in range(8):
                c_idx = lax.iota(jnp.int32, 16) + 16 * (m & 1)
                l_idx = jnp.full((16,), 4 * rr + (m >> 1), jnp.int32)
                vals = plsc.load_gather(blk[par], [c_idx, l_idx])
                rows[par][rr, pl.ds(m * 16, 16)] = vals
            return carry
        lax.fori_loop(0, 32, body, 0)

    pltpu.async_copy(rows[0], out_hbm.at[pl.ds(wid * 32, 32), :], wsem[0])
    pltpu.async_copy(rows[1], out_hbm.at[pl.ds((32 + wid) * 32, 32), :], wsem[1])

    def do_t2(t2, carry):
        for par in range(2):
            b = (2 * t2 + par) * NW + wid

            @pl.when(b < NBF)
            def _():
                pltpu.async_copy(
                    tableT_hbm.at[:, pl.ds(pl.multiple_of(b * 128, 128), 128)],
                    blk[par], rsem[par],
                )
        for par in range(2):
            b = (2 * t2 + par) * NW + wid

            @pl.when(b < NBF)
            def _():
                pltpu.make_async_copy(
                    tableT_hbm.at[:, pl.ds(0, 128)], blk[par], rsem[par]
                ).wait()
                pltpu.make_async_copy(
                    rows[par], out_hbm.at[pl.ds(0, 32), :], wsem[par]
                ).wait()
                transpose(par)
                pltpu.async_copy(
                    rows[par],
                    out_hbm.at[pl.ds(pl.multiple_of(b * 32, 32), 32), :],
                    wsem[par],
                )
        return carry

    lax.fori_loop(0, 123, do_t2, 0)
    pltpu.make_async_copy(rows[0], out_hbm.at[pl.ds(0, 32), :], wsem[0]).wait()
    pltpu.make_async_copy(rows[1], out_hbm.at[pl.ds(0, 32), :], wsem[1]).wait()


@functools.partial(
    pl.kernel,
    out_type=jax.ShapeDtypeStruct((NJ, DIM, NI), jnp.float32),
    mesh=plsc.VectorSubcoreMesh(core_axis_name="c", subcore_axis_name="s"),
    scratch_types=[
        pltpu.VMEM((8, IW), jnp.int32),
        pltpu.VMEM((HALF,), jnp.int32),
        pltpu.VMEM((HALF,), jnp.int32),
        pltpu.VMEM((HALF,), jnp.int32),
        pltpu.VMEM((HALF,), jnp.int32),
        pltpu.VMEM((HALF, 128), jnp.float32),
        pltpu.VMEM((HALF, 128), jnp.float32),
        pltpu.VMEM((DIM, IW), jnp.float32),
        pltpu.SemaphoreType.DMA,
        pltpu.SemaphoreType.DMA,
        pltpu.SemaphoreType.DMA,
    ],
    compiler_params=pltpu.CompilerParams(
        use_tc_tiling_on_sc=True, needs_layout_passes=False
    ),
)
def _emb_kernel(idx_hbm, table_hbm, out_hbm, idx_blk, bidx0, bidx1,
                qoff0, qoff1, big0, big1, outblk, gsem0, gsem1, wsem):
    wid = lax.axis_index("s") * NC + lax.axis_index("c")
    i0 = pl.multiple_of(wid * IW, 128)

    bidx = (bidx0, bidx1)
    qoff = (qoff0, qoff1)
    big = (big0, big1)
    gsem = (gsem0, gsem1)

    def prep(jj, h):
        def body(k, carry):
            v = idx_blk[jj, pl.ds(h * HALF + k * 16, 16)]
            bidx[h][pl.ds(k * 16, 16)] = lax.shift_right_logical(v, 2)
            qoff[h][pl.ds(k * 16, 16)] = lax.shift_left(v & 3, 5)
            return carry
        lax.fori_loop(0, HALF // 16, body, 0)

    def extract(h):
        def body(k, carry):
            row_v = lax.iota(jnp.int32, 16) + k * 16
            colb = qoff[h][pl.ds(k * 16, 16)]
            for c in range(DIM):
                vals = plsc.load_gather(big[h], [row_v, colb + c])
                outblk[c, pl.ds(h * HALF + k * 16, 16)] = vals
            return carry
        lax.fori_loop(0, HALF // 16, body, 0)

    pltpu.async_copy(outblk, out_hbm.at[0, :, pl.ds(i0, IW)], wsem)

    def do_jb(jb, carry):
        pltpu.sync_copy(
            idx_hbm.at[pl.ds(pl.multiple_of(jb * 8, 8), 8), pl.ds(i0, IW)],
            idx_blk,
        )

        def do_jj(jj, carry2):
            j = jb * 8 + jj

            @pl.when(j < NJ)
            def _():
                prep(jj, 0)
                pltpu.async_copy(table_hbm.at[bidx[0]], big[0], gsem[0])
                prep(jj, 1)
                pltpu.async_copy(table_hbm.at[bidx[1]], big[1], gsem[1])
                pltpu.make_async_copy(
                    outblk, out_hbm.at[0, :, pl.ds(i0, IW)], wsem
                ).wait()
                pltpu.make_async_copy(
                    table_hbm.at[bidx[0]], big[0], gsem[0]
                ).wait()
                extract(0)
                pltpu.make_async_copy(
                    table_hbm.at[bidx[1]], big[1], gsem[1]
                ).wait()
                extract(1)
                pltpu.async_copy(outblk, out_hbm.at[j, :, pl.ds(i0, IW)], wsem)

            return carry2

        lax.fori_loop(0, 8, do_jj, 0)
        return carry

    lax.fori_loop(0, NJP // 8, do_jb, 0)
    pltpu.make_async_copy(outblk, out_hbm.at[0, :, pl.ds(i0, IW)], wsem).wait()


def kernel(states, table):
    statesT = jnp.swapaxes(states, 0, 1)
    statesT_p = jnp.pad(statesT, ((0, NJP - NJ), (0, 0)))
    tableT = jnp.swapaxes(table, 0, 1)
    tail128 = table[NFULL:, :].reshape(NTAIL, 128)
    table128 = _relayout_kernel(tableT, tail128)
    outT = _emb_kernel(statesT_p, table128)
    return jnp.transpose(outT, (2, 0, 1))


# speedup vs baseline: 1.1487x; 1.0959x over previous
import functools

import jax
import jax.numpy as jnp
from jax import lax
from jax.experimental import pallas as pl
from jax.experimental.pallas import tpu as pltpu
from jax.experimental.pallas import tpu_sc as plsc

DIM = 32
NI = 16384
NJ = 50
NJP = 56
NBLK = 250000
NC, NS = 2, 16
NW = NC * NS
IW = NI // NW
HALF = 256


NFULL = (1000000 // 128) * 128
NBF = NFULL // 128
NTAIL = (1000000 - NFULL) * DIM // 128


@functools.partial(
    pl.kernel,
    out_type=jax.ShapeDtypeStruct((NBLK, 128), jnp.float32),
    mesh=plsc.VectorSubcoreMesh(core_axis_name="c", subcore_axis_name="s"),
    scratch_types=[
        pltpu.VMEM((DIM, 128), jnp.float32),
        pltpu.VMEM((DIM, 128), jnp.float32),
        pltpu.VMEM((32, 128), jnp.float32),
        pltpu.VMEM((32, 128), jnp.float32),
        pltpu.SemaphoreType.DMA,
        pltpu.SemaphoreType.DMA,
        pltpu.SemaphoreType.DMA,
        pltpu.SemaphoreType.DMA,
    ],
    compiler_params=pltpu.CompilerParams(
        use_tc_tiling_on_sc=True, needs_layout_passes=False
    ),
)
def _relayout_kernel(tableT_hbm, tail_hbm, out_hbm, blk0, blk1, rows0, rows1,
                     rsem0, rsem1, wsem0, wsem1):
    wid = lax.axis_index("s") * NC + lax.axis_index("c")
    blk = (blk0, blk1)
    rows = (rows0, rows1)
    rsem = (rsem0, rsem1)
    wsem = (wsem0, wsem1)

    @pl.when(wid == 0)
    def _():
        pltpu.sync_copy(tail_hbm, out_hbm.at[pl.ds(NBLK - NTAIL, NTAIL), :])

    lanes = lax.iota(jnp.int32, 16)
    riota = lax.shift_right_logical(lanes, 2)
    coff = lax.shift_left(lanes & 3, 5)

    def transpose(par):
        for mp in range(8):
            row_idx = riota + 4 * mp

            def body(c, carry):
                vals = blk[par][c, pl.ds(mp * 16, 16)]
                plsc.store_scatter(rows[par], [row_idx, coff + c], vals)
                return carry

            lax.fori_loop(0, 32, body, 0)

    pltpu.async_copy(rows[0], out_hbm.at[pl.ds(wid * 32, 32), :], wsem[0])
    pltpu.async_copy(rows[1], out_hbm.at[pl.ds((32 + wid) * 32, 32), :], wsem[1])

    def do_t2(t2, carry):
        for par in range(2):
            b = (2 * t2 + par) * NW + wid

            @pl.when(b < NBF)
            def _():
                pltpu.async_copy(
                    tableT_hbm.at[:, pl.ds(pl.multiple_of(b * 128, 128), 128)],
                    blk[par], rsem[par],
                )
        for par in range(2):
            b = (2 * t2 + par) * NW + wid

            @pl.when(b < NBF)
            def _():
                pltpu.make_async_copy(
                    tableT_hbm.at[:, pl.ds(0, 128)], blk[par], rsem[par]
                ).wait()
                pltpu.make_async_copy(
                    rows[par], out_hbm.at[pl.ds(0, 32), :], wsem[par]
                ).wait()
                transpose(par)
                pltpu.async_copy(
                    rows[par],
                    out_hbm.at[pl.ds(pl.multiple_of(b * 32, 32), 32), :],
                    wsem[par],
                )
        return carry

    lax.fori_loop(0, 123, do_t2, 0)
    pltpu.make_async_copy(rows[0], out_hbm.at[pl.ds(0, 32), :], wsem[0]).wait()
    pltpu.make_async_copy(rows[1], out_hbm.at[pl.ds(0, 32), :], wsem[1]).wait()


@functools.partial(
    pl.kernel,
    out_type=jax.ShapeDtypeStruct((NJ, DIM, NI), jnp.float32),
    mesh=plsc.VectorSubcoreMesh(core_axis_name="c", subcore_axis_name="s"),
    scratch_types=[
        pltpu.VMEM((8, IW), jnp.int32),
        pltpu.VMEM((HALF,), jnp.int32),
        pltpu.VMEM((HALF,), jnp.int32),
        pltpu.VMEM((HALF,), jnp.int32),
        pltpu.VMEM((HALF,), jnp.int32),
        pltpu.VMEM((HALF, 128), jnp.float32),
        pltpu.VMEM((HALF, 128), jnp.float32),
        pltpu.VMEM((DIM, IW), jnp.float32),
        pltpu.SemaphoreType.DMA,
        pltpu.SemaphoreType.DMA,
        pltpu.SemaphoreType.DMA,
    ],
    compiler_params=pltpu.CompilerParams(
        use_tc_tiling_on_sc=True, needs_layout_passes=False
    ),
)
def _emb_kernel(idx_hbm, table_hbm, out_hbm, idx_blk, bidx0, bidx1,
                qoff0, qoff1, big0, big1, outblk, gsem0, gsem1, wsem):
    wid = lax.axis_index("s") * NC + lax.axis_index("c")
    i0 = pl.multiple_of(wid * IW, 128)

    bidx = (bidx0, bidx1)
    qoff = (qoff0, qoff1)
    big = (big0, big1)
    gsem = (gsem0, gsem1)

    def prep(jj, h):
        def body(k, carry):
            v = idx_blk[jj, pl.ds(h * HALF + k * 16, 16)]
            bidx[h][pl.ds(k * 16, 16)] = lax.shift_right_logical(v, 2)
            qoff[h][pl.ds(k * 16, 16)] = lax.shift_left(v & 3, 5)
            return carry
        lax.fori_loop(0, HALF // 16, body, 0)

    def extract(h):
        def body(k, carry):
            row_v = lax.iota(jnp.int32, 16) + k * 16
            colb = qoff[h][pl.ds(k * 16, 16)]
            for c in range(DIM):
                vals = plsc.load_gather(big[h], [row_v, colb + c])
                outblk[c, pl.ds(h * HALF + k * 16, 16)] = vals
            return carry
        lax.fori_loop(0, HALF // 16, body, 0)

    pltpu.async_copy(outblk, out_hbm.at[0, :, pl.ds(i0, IW)], wsem)

    def do_jb(jb, carry):
        pltpu.sync_copy(
            idx_hbm.at[pl.ds(pl.multiple_of(jb * 8, 8), 8), pl.ds(i0, IW)],
            idx_blk,
        )

        def do_jj(jj, carry2):
            j = jb * 8 + jj

            @pl.when(j < NJ)
            def _():
                prep(jj, 0)
                pltpu.async_copy(table_hbm.at[bidx[0]], big[0], gsem[0])
                prep(jj, 1)
                pltpu.async_copy(table_hbm.at[bidx[1]], big[1], gsem[1])
                pltpu.make_async_copy(
                    outblk, out_hbm.at[0, :, pl.ds(i0, IW)], wsem
                ).wait()
                pltpu.make_async_copy(
                    table_hbm.at[bidx[0]], big[0], gsem[0]
                ).wait()
                extract(0)
                pltpu.make_async_copy(
                    table_hbm.at[bidx[1]], big[1], gsem[1]
                ).wait()
                extract(1)
                pltpu.async_copy(outblk, out_hbm.at[j, :, pl.ds(i0, IW)], wsem)

            return carry2

        lax.fori_loop(0, 8, do_jj, 0)
        return carry

    lax.fori_loop(0, NJP // 8, do_jb, 0)
    pltpu.make_async_copy(outblk, out_hbm.at[0, :, pl.ds(i0, IW)], wsem).wait()


def kernel(states, table):
    statesT = jnp.swapaxes(states, 0, 1)
    statesT_p = jnp.pad(statesT, ((0, NJP - NJ), (0, 0)))
    tableT = jnp.swapaxes(table, 0, 1)
    tail128 = table[NFULL:, :].reshape(NTAIL, 128)
    table128 = _relayout_kernel(tableT, tail128)
    outT = _emb_kernel(statesT_p, table128)
    return jnp.transpose(outT, (2, 0, 1))


# speedup vs baseline: 1.1744x; 1.0223x over previous
import functools

import jax
import jax.numpy as jnp
from jax import lax
from jax.experimental import pallas as pl
from jax.experimental.pallas import tpu as pltpu
from jax.experimental.pallas import tpu_sc as plsc

DIM = 32
NI = 16384
NJ = 50
NJP = 56
NBLK = 250000
NC, NS = 2, 16
NW = NC * NS
IW = NI // NW
HALF = 256


NFULL = (1000000 // 512) * 512
NSLAB = NFULL // 512
NTAIL = (1000000 - NFULL) * DIM // 128


@functools.partial(
    pl.kernel,
    out_type=jax.ShapeDtypeStruct((NBLK, 128), jnp.float32),
    mesh=plsc.VectorSubcoreMesh(core_axis_name="c", subcore_axis_name="s"),
    scratch_types=[
        pltpu.VMEM((DIM, 512), jnp.float32),
        pltpu.VMEM((DIM, 512), jnp.float32),
        pltpu.VMEM((128, 128), jnp.float32),
        pltpu.VMEM((128, 128), jnp.float32),
        pltpu.SemaphoreType.DMA,
        pltpu.SemaphoreType.DMA,
        pltpu.SemaphoreType.DMA,
        pltpu.SemaphoreType.DMA,
    ],
    compiler_params=pltpu.CompilerParams(
        use_tc_tiling_on_sc=True, needs_layout_passes=False
    ),
)
def _relayout_kernel(tableT_hbm, tail_hbm, out_hbm, blk0, blk1, rows0, rows1,
                     rsem0, rsem1, wsem0, wsem1):
    wid = lax.axis_index("s") * NC + lax.axis_index("c")
    blk = (blk0, blk1)
    rows = (rows0, rows1)
    rsem = (rsem0, rsem1)
    wsem = (wsem0, wsem1)

    @pl.when(wid == 0)
    def _():
        pltpu.sync_copy(tail_hbm, out_hbm.at[pl.ds(NBLK - NTAIL, NTAIL), :])

    lanes = lax.iota(jnp.int32, 16)
    riota = lax.shift_right_logical(lanes, 2)
    coff = lax.shift_left(lanes & 3, 5)

    def transpose(par):
        for sb in range(4):
            def body(mp, carry, sb=sb):
                row_idx = riota + 4 * mp + 32 * sb
                for c in range(DIM):
                    vals = blk[par][c, pl.ds(sb * 128 + mp * 16, 16)]
                    plsc.store_scatter(rows[par], [row_idx, coff + c], vals)
                return carry

            lax.fori_loop(0, 8, body, 0)

    pltpu.async_copy(rows[0], out_hbm.at[pl.ds(wid * 128, 128), :], wsem[0])
    pltpu.async_copy(rows[1], out_hbm.at[pl.ds((NW + wid) * 128, 128), :], wsem[1])

    def do_t2(t2, carry):
        for par in range(2):
            b = (2 * t2 + par) * NW + wid

            @pl.when(b < NSLAB)
            def _():
                pltpu.async_copy(
                    tableT_hbm.at[:, pl.ds(pl.multiple_of(b * 512, 128), 512)],
                    blk[par], rsem[par],
                )
        for par in range(2):
            b = (2 * t2 + par) * NW + wid

            @pl.when(b < NSLAB)
            def _():
                pltpu.make_async_copy(
                    tableT_hbm.at[:, pl.ds(0, 512)], blk[par], rsem[par]
                ).wait()
                pltpu.make_async_copy(
                    rows[par], out_hbm.at[pl.ds(0, 128), :], wsem[par]
                ).wait()
                transpose(par)
                pltpu.async_copy(
                    rows[par],
                    out_hbm.at[pl.ds(pl.multiple_of(b * 128, 128), 128), :],
                    wsem[par],
                )
        return carry

    lax.fori_loop(0, 31, do_t2, 0)
    pltpu.make_async_copy(rows[0], out_hbm.at[pl.ds(0, 128), :], wsem[0]).wait()
    pltpu.make_async_copy(rows[1], out_hbm.at[pl.ds(0, 128), :], wsem[1]).wait()


@functools.partial(
    pl.kernel,
    out_type=jax.ShapeDtypeStruct((NJ, DIM, NI), jnp.float32),
    mesh=plsc.VectorSubcoreMesh(core_axis_name="c", subcore_axis_name="s"),
    scratch_types=[
        pltpu.VMEM((8, IW), jnp.int32),
        pltpu.VMEM((HALF,), jnp.int32),
        pltpu.VMEM((HALF,), jnp.int32),
        pltpu.VMEM((HALF,), jnp.int32),
        pltpu.VMEM((HALF,), jnp.int32),
        pltpu.VMEM((HALF, 128), jnp.float32),
        pltpu.VMEM((HALF, 128), jnp.float32),
        pltpu.VMEM((DIM, IW), jnp.float32),
        pltpu.SemaphoreType.DMA,
        pltpu.SemaphoreType.DMA,
        pltpu.SemaphoreType.DMA,
    ],
    compiler_params=pltpu.CompilerParams(
        use_tc_tiling_on_sc=True, needs_layout_passes=False
    ),
)
def _emb_kernel(idx_hbm, table_hbm, out_hbm, idx_blk, bidx0, bidx1,
                qoff0, qoff1, big0, big1, outblk, gsem0, gsem1, wsem):
    wid = lax.axis_index("s") * NC + lax.axis_index("c")
    i0 = pl.multiple_of(wid * IW, 128)

    bidx = (bidx0, bidx1)
    qoff = (qoff0, qoff1)
    big = (big0, big1)
    gsem = (gsem0, gsem1)

    def prep(jj, h):
        def body(k, carry):
            v = idx_blk[jj, pl.ds(h * HALF + k * 16, 16)]
            bidx[h][pl.ds(k * 16, 16)] = lax.shift_right_logical(v, 2)
            qoff[h][pl.ds(k * 16, 16)] = lax.shift_left(v & 3, 5)
            return carry
        lax.fori_loop(0, HALF // 16, body, 0)

    def extract(h):
        def body(k, carry):
            row_v = lax.iota(jnp.int32, 16) + k * 16
            colb = qoff[h][pl.ds(k * 16, 16)]
            for c in range(DIM):
                vals = plsc.load_gather(big[h], [row_v, colb + c])
                outblk[c, pl.ds(h * HALF + k * 16, 16)] = vals
            return carry
        lax.fori_loop(0, HALF // 16, body, 0)

    pltpu.async_copy(outblk, out_hbm.at[0, :, pl.ds(i0, IW)], wsem)

    def do_jb(jb, carry):
        pltpu.sync_copy(
            idx_hbm.at[pl.ds(pl.multiple_of(jb * 8, 8), 8), pl.ds(i0, IW)],
            idx_blk,
        )

        def do_jj(jj, carry2):
            j = jb * 8 + jj

            @pl.when(j < NJ)
            def _():
                prep(jj, 0)
                pltpu.async_copy(table_hbm.at[bidx[0]], big[0], gsem[0])
                prep(jj, 1)
                pltpu.async_copy(table_hbm.at[bidx[1]], big[1], gsem[1])
                pltpu.make_async_copy(
                    outblk, out_hbm.at[0, :, pl.ds(i0, IW)], wsem
                ).wait()
                pltpu.make_async_copy(
                    table_hbm.at[bidx[0]], big[0], gsem[0]
                ).wait()
                extract(0)
                pltpu.make_async_copy(
                    table_hbm.at[bidx[1]], big[1], gsem[1]
                ).wait()
                extract(1)
                pltpu.async_copy(outblk, out_hbm.at[j, :, pl.ds(i0, IW)], wsem)

            return carry2

        lax.fori_loop(0, 8, do_jj, 0)
        return carry

    lax.fori_loop(0, NJP // 8, do_jb, 0)
    pltpu.make_async_copy(outblk, out_hbm.at[0, :, pl.ds(i0, IW)], wsem).wait()


def kernel(states, table):
    statesT = jnp.swapaxes(states, 0, 1)
    statesT_p = jnp.pad(statesT, ((0, NJP - NJ), (0, 0)))
    tableT = jnp.swapaxes(table, 0, 1)
    tail128 = table[NFULL:, :].reshape(NTAIL, 128)
    table128 = _relayout_kernel(tableT, tail128)
    outT = _emb_kernel(statesT_p, table128)
    return jnp.transpose(outT, (2, 0, 1))
